# Initial kernel scaffold; baseline (speedup 1.0000x reference)
import jax
import jax.numpy as jnp
from jax.experimental import pallas as pl


def kernel(node_features, edge_index, current_partition, gW1, gb1, gW2, gb2, nsW1, nsb1, nsW2, nsb2, psW1, psb1, psW2, psb2, cW1, cb1, cW2, cb2):
    raise NotImplementedError("write your pallas kernel here")


# speedup vs baseline: 1.9322x; 1.9322x over previous
import functools

import jax
import jax.numpy as jnp
from jax import lax
from jax.experimental import pallas as pl
from jax.experimental.pallas import tpu as pltpu
from jax.experimental.pallas import tpu_sc as plsc

_N = 10000
_E = 320000
_D = 128
_P = 16

_ROWS = 10240
_TRASH = 10000
_ACC_ROWS = _ROWS
_ZCHUNK = _ACC_ROWS // 16
_CHUNK = 128
_NW = 32
_CW = 79
_EPAD = _NW * _CW * _CHUNK
_NCHUNKS = _NW * _CW
_GRID = _ROWS // 128

_f32 = jnp.float32


def _sc_agg_body(x_hbm, src_hbm, dst_hbm, z_hbm, s_out,
                 src_v, dst_v, rows_v, acc, sem):
    c = lax.axis_index("c")
    s = lax.axis_index("s")
    w = c * 16 + s
    pltpu.sync_copy(z_hbm, acc.at[pl.ds(s * _ZCHUNK, _ZCHUNK)])
    pltpu.sync_copy(src_hbm.at[w], src_v)
    pltpu.sync_copy(dst_hbm.at[w], dst_v)
    plsc.subcore_barrier()

    def step(j, carry):
        pltpu.async_copy(x_hbm.at[src_v.at[j]], rows_v, sem).wait()
        pltpu.sync_copy(rows_v, acc.at[dst_v.at[j]], add=True)
        return carry

    lax.fori_loop(0, _CW, step, 0)
    plsc.subcore_barrier()
    o = s * _ZCHUNK
    pltpu.sync_copy(acc.at[pl.ds(o, _ZCHUNK)], s_out.at[c, pl.ds(o, _ZCHUNK)])


@functools.cache
def _sc_agg():
    mesh = plsc.VectorSubcoreMesh(core_axis_name="c", subcore_axis_name="s")
    return pl.kernel(
        _sc_agg_body,
        mesh=mesh,
        out_type=[jax.ShapeDtypeStruct((2, _ROWS, _D), _f32)],
        scratch_types=[
            pltpu.VMEM((_CW, _CHUNK), jnp.int32),
            pltpu.VMEM((_CW, _CHUNK), jnp.int32),
            pltpu.VMEM((_CHUNK, _D), _f32),
            pltpu.VMEM_SHARED((_ACC_ROWS, _D), _f32),
            pltpu.SemaphoreType.DMA,
        ],
    )


def _sc_deg_body(dst_hbm, z_hbm, ones_hbm, d_out,
                 dst_v, ones_v, dacc):
    c = lax.axis_index("c")
    s = lax.axis_index("s")
    w = c * 16 + s
    pltpu.sync_copy(z_hbm, dacc.at[pl.ds(s * _ZCHUNK, _ZCHUNK)])
    pltpu.sync_copy(ones_hbm, ones_v)
    pltpu.sync_copy(dst_hbm.at[w], dst_v)
    plsc.subcore_barrier()

    def step(j, carry):
        pltpu.sync_copy(ones_v, dacc.at[dst_v.at[j]], add=True)
        return carry

    lax.fori_loop(0, _CW, step, 0)
    plsc.subcore_barrier()
    o = s * _ZCHUNK
    pltpu.sync_copy(dacc.at[pl.ds(o, _ZCHUNK)], d_out.at[c, pl.ds(o, _ZCHUNK)])


@functools.cache
def _sc_deg():
    mesh = plsc.VectorSubcoreMesh(core_axis_name="c", subcore_axis_name="s")
    return pl.kernel(
        _sc_deg_body,
        mesh=mesh,
        out_type=[jax.ShapeDtypeStruct((2, _ROWS, _D), _f32)],
        scratch_types=[
            pltpu.VMEM((_CW, _CHUNK), jnp.int32),
            pltpu.VMEM((_CHUNK, _D), _f32),
            pltpu.VMEM_SHARED((_ACC_ROWS, _D), _f32),
        ],
    )


def _tc1_body(sp0, sp1, dp0, dp1, xp, w1, b1, part, x1o, cnto):
    i = pl.program_id(0)
    ssum = sp0[0] + sp1[0] + xp[...]
    dcnt = dp0[0][:, 0:1] + dp1[0][:, 0:1]
    z = ssum / (dcnt + 1.0)
    y = jnp.dot(z, w1[...], preferred_element_type=_f32) + b1[...]
    x1o[...] = jnp.maximum(y, 0.0)
    pids = lax.broadcasted_iota(jnp.int32, (_P, 128), 0)
    cmp = (part[0] == pids).astype(_f32)

    @pl.when(i == 0)
    def _():
        cnto[...] = cmp

    @pl.when(i != 0)
    def _():
        cnto[...] = cnto[...] + cmp


def _tc2_body(sp0, sp1, dp0, dp1, xp, w2, b2, nsw1, nsb1, nsw2, nsb2,
              psw1, psb1, psw2, psb2, cw1, cb1, cw2, cb2, cnt, part,
              nlo, pmlo, valo, acc):
    i = pl.program_id(0)
    ssum = sp0[0] + sp1[0] + xp[...]
    dcnt = dp0[0][:, 0:1] + dp1[0][:, 0:1]
    x2 = jnp.dot(ssum / (dcnt + 1.0), w2[...],
                 preferred_element_type=_f32) + b2[...]
    h = jnp.maximum(jnp.dot(x2, nsw1[...], preferred_element_type=_f32)
                    + nsb1[...], 0.0)
    nlo[...] = jnp.dot(h, nsw2[...], preferred_element_type=_f32) + nsb2[...]
    hp = jnp.maximum(jnp.dot(x2, psw1[...], preferred_element_type=_f32)
                     + psb1[...], 0.0)
    plog = jnp.dot(hp, psw2[...], preferred_element_type=_f32) + psb2[...]
    onehot = part[...] == lax.broadcasted_iota(jnp.int32, (128, _P), 1)
    lockedsel = onehot & (cnt[...] <= 1.0)
    locked = jnp.sum(lockedsel.astype(_f32), axis=1, keepdims=True) > 0.0
    mask = locked & jnp.logical_not(onehot)
    pmlo[...] = jnp.where(mask, -jnp.inf, plog)
    rowid = i * 128 + lax.broadcasted_iota(jnp.int32, (128, 1), 0)
    xs = jnp.sum(jnp.where(rowid < _N, x2, 0.0), axis=0, keepdims=True)

    @pl.when(i == 0)
    def _():
        acc[...] = jnp.zeros_like(acc)

    acc[...] = acc[...] + jnp.broadcast_to(xs, (8, 128))

    @pl.when(i == pl.num_programs(0) - 1)
    def _():
        mean = acc[0:1, :] / float(_N)
        hv = jnp.maximum(jnp.dot(mean, cw1[...], preferred_element_type=_f32)
                         + cb1[...], 0.0)
        v = jnp.dot(hv, cw2[...], preferred_element_type=_f32) + cb2[...]
        valo[...] = jnp.broadcast_to(v, (8, 128))


def _full(shape):
    return pl.BlockSpec(shape, lambda i: tuple(0 for _ in shape))


@functools.cache
def _tc1():
    return pl.pallas_call(
        _tc1_body,
        grid=(_GRID,),
        in_specs=[
            pl.BlockSpec((1, 128, _D), lambda i: (0, i, 0)),
            pl.BlockSpec((1, 128, _D), lambda i: (1, i, 0)),
            pl.BlockSpec((1, 128, _D), lambda i: (0, i, 0)),
            pl.BlockSpec((1, 128, _D), lambda i: (1, i, 0)),
            pl.BlockSpec((128, _D), lambda i: (i, 0)),
            _full((_D, _D)),
            _full((1, _D)),
            pl.BlockSpec((1, 1, 128), lambda i: (i, 0, 0)),
        ],
        out_specs=[
            pl.BlockSpec((128, _D), lambda i: (i, 0)),
            pl.BlockSpec((_P, 128), lambda i: (0, 0)),
        ],
        out_shape=[
            jax.ShapeDtypeStruct((_ROWS, _D), _f32),
            jax.ShapeDtypeStruct((_P, 128), _f32),
        ],
    )


@functools.cache
def _tc2():
    return pl.pallas_call(
        _tc2_body,
        grid=(_GRID,),
        in_specs=[
            pl.BlockSpec((1, 128, _D), lambda i: (0, i, 0)),
            pl.BlockSpec((1, 128, _D), lambda i: (1, i, 0)),
            pl.BlockSpec((1, 128, _D), lambda i: (0, i, 0)),
            pl.BlockSpec((1, 128, _D), lambda i: (1, i, 0)),
            pl.BlockSpec((128, _D), lambda i: (i, 0)),
            _full((_D, _D)),
            _full((1, _D)),
            _full((_D, 64)),
            _full((1, 64)),
            _full((64, 1)),
            _full((1, 1)),
            _full((_D, 64)),
            _full((1, 64)),
            _full((64, _P)),
            _full((1, _P)),
            _full((_D, _D)),
            _full((1, _D)),
            _full((_D, 1)),
            _full((1, 1)),
            _full((1, _P)),
            pl.BlockSpec((128, 1), lambda i: (i, 0)),
        ],
        out_specs=[
            pl.BlockSpec((128, 1), lambda i: (i, 0)),
            pl.BlockSpec((128, _P), lambda i: (i, 0)),
            pl.BlockSpec((8, 128), lambda i: (0, 0)),
        ],
        out_shape=[
            jax.ShapeDtypeStruct((_ROWS, 1), _f32),
            jax.ShapeDtypeStruct((_ROWS, _P), _f32),
            jax.ShapeDtypeStruct((8, 128), _f32),
        ],
        scratch_shapes=[pltpu.VMEM((8, 128), _f32)],
    )


def _agg(x_pad, src_c, dst_c, zrows):
    return _sc_agg()(x_pad, src_c, dst_c, zrows)


def _deg(dst_c, zrows, ones):
    return _sc_deg()(dst_c, zrows, ones)


def kernel(node_features, edge_index, current_partition,
           gW1, gb1, gW2, gb2,
           nsW1, nsb1, nsW2, nsb2,
           psW1, psb1, psW2, psb2,
           cW1, cb1, cW2, cb2):
    key = edge_index[0] * _N + edge_index[1]
    skey = jnp.sort(key)
    dup = jnp.concatenate(
        [jnp.zeros((1,), jnp.bool_), skey[1:] == skey[:-1]])
    dst = skey // _N
    src = skey - dst * _N
    dst = jnp.where(dup, _TRASH, dst)
    pad = _EPAD - _E
    src_c = jnp.concatenate(
        [src, jnp.zeros((pad,), jnp.int32)]).reshape(_NW, _CW, _CHUNK)
    dst_c = jnp.concatenate(
        [dst, jnp.full((pad,), _TRASH, jnp.int32)]).reshape(_NW, _CW, _CHUNK)

    x_pad = jnp.pad(node_features, ((0, _ROWS - _N), (0, 0)))
    zrows = jnp.zeros((_ZCHUNK, _D), _f32)
    ones = jnp.ones((_CHUNK, _D), _f32)
    part_pad = jnp.pad(current_partition, (0, _ROWS - _N),
                       constant_values=-1)
    part3 = part_pad.reshape(_GRID, 1, 128)
    part2 = part_pad.reshape(_ROWS, 1)

    (s1p,) = _agg(x_pad, src_c, dst_c, zrows)
    (d1p,) = _deg(dst_c, zrows, ones)
    x1, counts = _tc1()(s1p, s1p, d1p, d1p, x_pad, gW1,
                        gb1.reshape(1, _D), part3)
    (s2p,) = _agg(x1, src_c, dst_c, zrows)
    cnt16 = jnp.sum(counts, axis=1).reshape(1, _P)
    nl, pml, val = _tc2()(s2p, s2p, d1p, d1p, x1, gW2, gb2.reshape(1, _D),
                          nsW1, nsb1.reshape(1, 64), nsW2,
                          nsb2.reshape(1, 1),
                          psW1, psb1.reshape(1, 64), psW2,
                          psb2.reshape(1, _P),
                          cW1, cb1.reshape(1, _D), cW2, cb2.reshape(1, 1),
                          cnt16, part2)
    return (nl[:_N, 0], pml[:_N], val[0, 0])
